# baseline (speedup 1.0000x reference)
import functools

import jax
import jax.numpy as jnp
from jax import lax
from jax.experimental import pallas as pl
from jax.experimental.pallas import tpu as pltpu
from jax.experimental.pallas import tpu_sc as plsc

M = 20000
B = 4096
D = 3 * 32 * 32
NC, NS, L = 2, 16, 16
NW = NC * NS
LW = 25
LROWS = M // LW
BIG = 1 << 19
HUGE = 1 << 30
LSZ = B + 2 * L
NBLK = (M + 127) // 128
PMAX = 64
CMSEG = 5120
SBL = 176


def _sc_body(lbl_in, y_in, idx_in, x2_in, lbl_out, p_out, cm_out,
             winner_v, idx_v, y_v, lbl_v, shift_v, rows_l, src_l,
             counts_v, starts_v, cm_v, zbuf, gbuf, zsems, gsem):
    wid = lax.axis_index("s") * NC + lax.axis_index("c")
    lane = lax.iota(jnp.int32, L)

    pltpu.sync_copy(idx_in, idx_v)
    shift_v[pl.ds(L, L)] = jnp.full((L,), HUGE, jnp.int32)

    def init_body(i, c):
        winner_v[pl.ds(i * L, L)] = jnp.full((L,), -1, jnp.int32)
        return c
    lax.fori_loop(0, (M + L) // L, init_body, 0)

    def scan_body(v, c):
        vec = idx_v[pl.ds(v * L, L)]
        valid = vec < M
        key = jnp.where(valid, vec * L + lane, BIG + lane)
        skey, slane = plsc.sort_key_val(key, lane)
        shift_v[pl.ds(0, L)] = skey
        nkey = shift_v[pl.ds(1, L)]
        keep = ((skey >> 4) != (nkey >> 4)) & (skey < BIG)
        plsc.store_scatter(winner_v, [skey >> 4], v * L + slane, mask=keep)
        return c
    lax.fori_loop(0, B // L, scan_body, 0)

    @pl.when(wid < LW)
    def _labels():
        pltpu.sync_copy(y_in, y_v)
        l0 = wid * LROWS
        pltpu.sync_copy(lbl_in.at[pl.ds(l0, LROWS)], lbl_v)

        def lbl_body(v, c):
            wv = winner_v[pl.ds(l0 + v * L, L)]
            m = wv >= 0
            yv = plsc.load_gather(y_v, [jnp.maximum(wv, 0)])
            cur = lbl_v[pl.ds(v * L, L)]
            lbl_v[pl.ds(v * L, L)] = jnp.where(m, yv, cur)
            return c
        lax.fori_loop(0, LROWS // L, lbl_body, 0)
        pltpu.sync_copy(lbl_v, lbl_out.at[pl.ds(l0, LROWS)])

    def cmp_body(g, base):
        wv = winner_v[pl.ds(g * L, L)]
        m = wv >= 0
        pc = plsc.cumsum(jnp.where(m, 1, 0))
        pos = base + pc - 1
        plsc.store_scatter(rows_l, [pos], g * L + lane, mask=m)
        plsc.store_scatter(src_l, [pos], wv, mask=m)
        return base + pc[L - 1]
    cnt = lax.fori_loop(0, M // L, cmp_body, jnp.int32(0))
    nt = (cnt + L - 1) // L

    def cinit(i, c):
        counts_v[pl.ds(i * L, L)] = jnp.zeros((L,), jnp.int32)
        return c
    lax.fori_loop(0, SBL // L, cinit, 0)

    def hist_body(t, c):
        rv = rows_l[pl.ds(t * L, L)]
        m = (t * L + lane) < cnt
        jv = jnp.clip(rv >> 7, 0, NBLK - 1)
        plsc.addupdate_scatter(counts_v, [jv], jnp.where(m, 1, 0), mask=m)
        return c
    lax.fori_loop(0, nt, hist_body, 0)

    def scan2(v, carry):
        cv = counts_v[pl.ds(v * L, L)]
        cs = plsc.cumsum(cv)
        starts_v[pl.ds(v * L, L)] = cs - cv + carry
        return carry + cs[L - 1]
    lax.fori_loop(0, SBL // L, scan2, jnp.int32(0))

    def zb(i, c):
        for r in range(8):
            zbuf[r, pl.ds(i * L, L)] = jnp.zeros((L,), jnp.float32)
        return c
    lax.fori_loop(0, D // L, zb, 0)

    def pfill(jj, c):
        j = wid + NW * jj

        @pl.when(j < NBLK)
        def _fill(j=j):
            sv = starts_v[pl.ds(j, L)]
            s_j = sv[0]
            nw = sv[1] - sv[0]
            for q in range(PMAX // 8):
                pltpu.async_copy(
                    zbuf, p_out.at[pl.ds(j * PMAX + q * 8, 8)],
                    zsems[q % (PMAX // L)])
            for q in range(PMAX // 8):
                pltpu.make_async_copy(
                    zbuf, p_out.at[pl.ds(j * PMAX + q * 8, 8)],
                    zsems[q % (PMAX // L)]).wait()
            for t in range(PMAX // L):

                @pl.when(t * L < nw)
                def _chunk(t=t, s_j=s_j, nw=nw, j=j):
                    slot = t * L + lane
                    m = slot < nw
                    srcv = src_l[pl.ds(s_j + t * L, L)]
                    srcc = jnp.clip(jnp.where(m, srcv, 0), 0, B - 1)
                    dstv = j * PMAX + jnp.where(m, slot, PMAX - 1)
                    pltpu.async_copy(x2_in.at[srcc], gbuf, gsem)
                    pltpu.make_async_copy(x2_in.at[srcc], gbuf, gsem).wait()
                    pltpu.async_copy(gbuf, p_out.at[dstv], gsem)
                    pltpu.make_async_copy(gbuf, p_out.at[dstv], gsem).wait()
        return c
    lax.fori_loop(0, (NBLK + NW - 1) // NW, pfill, 0)

    @pl.when(wid == 0)
    def _colmap():
        for seg in range(4):
            def cminit(i, c):
                cm_v[pl.ds(i * L, L)] = jnp.full((L,), -1, jnp.int32)
                return c
            lax.fori_loop(0, CMSEG // L, cminit, 0)

            def cmb(t, c, seg=seg):
                rv = rows_l[pl.ds(t * L, L)]
                kv = t * L + lane
                m = kv < cnt
                jv = jnp.clip(rv >> 7, 0, NBLK - 1)
                sjv = plsc.load_gather(starts_v, [jv])
                slotv = jnp.clip(kv - sjv, 0, PMAX - 1)
                loc = rv - seg * CMSEG
                m2 = m & (loc >= 0) & (loc < CMSEG)
                plsc.store_scatter(cm_v, [jnp.clip(loc, 0, CMSEG - 1)],
                                   slotv, mask=m2)
                return c
            lax.fori_loop(0, nt, cmb, 0)
            pltpu.sync_copy(cm_v, cm_out.at[pl.ds(seg * CMSEG, CMSEG)])


@functools.cache
def _build_sc():
    mesh = plsc.VectorSubcoreMesh(core_axis_name="c", subcore_axis_name="s",
                                  num_cores=NC, num_subcores=NS)
    return pl.kernel(
        _sc_body,
        out_type=(jax.ShapeDtypeStruct((M,), jnp.int32),
                  jax.ShapeDtypeStruct((NBLK * PMAX, D), jnp.float32),
                  jax.ShapeDtypeStruct((4 * CMSEG,), jnp.int32)),
        mesh=mesh,
        compiler_params=pltpu.CompilerParams(use_tc_tiling_on_sc=False,
                                             needs_layout_passes=False),
        scratch_types=dict(
            winner_v=pltpu.VMEM((M + L,), jnp.int32),
            idx_v=pltpu.VMEM((B,), jnp.int32),
            y_v=pltpu.VMEM((B,), jnp.int32),
            lbl_v=pltpu.VMEM((LROWS,), jnp.int32),
            shift_v=pltpu.VMEM((2 * L,), jnp.int32),
            rows_l=pltpu.VMEM((LSZ,), jnp.int32),
            src_l=pltpu.VMEM((LSZ,), jnp.int32),
            counts_v=pltpu.VMEM((SBL,), jnp.int32),
            starts_v=pltpu.VMEM((SBL,), jnp.int32),
            cm_v=pltpu.VMEM((CMSEG,), jnp.int32),
            zbuf=pltpu.VMEM((8, D), jnp.float32),
            gbuf=pltpu.VMEM((L, D), jnp.float32),
            zsems=[pltpu.SemaphoreType.DMA for _ in range(PMAX // L)],
            gsem=pltpu.SemaphoreType.DMA,
        ),
    )


def _merge_body(img_ref, p_ref, cm_ref, out_ref):
    i = pl.program_id(0)
    v = img_ref[...].reshape(D, 128)
    cmrow = cm_ref[pl.ds(lax.rem(i, 8), 1), :]
    oh = (lax.broadcasted_iota(jnp.int32, (PMAX, 128), 0)
          == cmrow).astype(jnp.float32)
    patch = lax.dot_general(p_ref[...], oh, (((0,), (0,)), ((), ())),
                            preferred_element_type=jnp.float32)
    keep = jnp.broadcast_to(cmrow >= 0, (D, 128))
    out = jnp.where(keep, patch, v)
    out_ref[...] = out.reshape(3, 32, 32, 128)


@functools.cache
def _build_merge():
    return pl.pallas_call(
        _merge_body,
        grid=(NBLK,),
        in_specs=[
            pl.BlockSpec((3, 32, 32, 128), lambda i: (0, 0, 0, i)),
            pl.BlockSpec((PMAX, D), lambda i: (i, 0)),
            pl.BlockSpec((8, 128), lambda i: (i // 8, 0)),
        ],
        out_specs=pl.BlockSpec((3, 32, 32, 128), lambda i: (0, 0, 0, i)),
        out_shape=jax.ShapeDtypeStruct((3, 32, 32, M), jnp.float32),
    )


def kernel(buffer_img, buffer_label, x, y, idx):
    out_lbl, p_mat, cm = _build_sc()(buffer_label, y, idx, x.reshape(B, D))
    cm2 = cm.reshape(160, 128)
    out_t = _build_merge()(jnp.transpose(buffer_img, (1, 2, 3, 0)),
                           p_mat, cm2)
    return jnp.transpose(out_t, (3, 0, 1, 2)), out_lbl


# speedup vs baseline: 1.2436x; 1.2436x over previous
import functools

import jax
import jax.numpy as jnp
from jax import lax
from jax.experimental import pallas as pl
from jax.experimental.pallas import tpu as pltpu
from jax.experimental.pallas import tpu_sc as plsc

M = 20000
B = 4096
IMG = (3, 32, 32)
D = 3072
NC, NS, L = 2, 16, 16
NW = NC * NS
LW = 25
LROWS = M // LW
BIG = 1 << 19
HUGE = 1 << 30
LSZ = B + 2 * L
CBT = 512
PS = 16


def _sc_body(lbl_in, y_in, idx_in, lbl_out, rows_out, src_out, cnt_out,
             winner_v, idx_v, y_v, lbl_v, shift_v, rows_l, src_l):
    wid = lax.axis_index("s") * NC + lax.axis_index("c")

    pltpu.sync_copy(idx_in, idx_v)

    lane = lax.iota(jnp.int32, L)
    shift_v[pl.ds(L, L)] = jnp.full((L,), HUGE, jnp.int32)

    @pl.when(wid < LW)
    def _scan():
        def init_body(i, c):
            winner_v[pl.ds(i * L, L)] = jnp.full((L,), -1, jnp.int32)
            return c
        lax.fori_loop(0, (M + L) // L, init_body, 0)

        def scan_body(v, c):
            vec = idx_v[pl.ds(v * L, L)]
            valid = vec < M
            key = jnp.where(valid, vec * L + lane, BIG + lane)
            skey, slane = plsc.sort_key_val(key, lane)
            shift_v[pl.ds(0, L)] = skey
            nkey = shift_v[pl.ds(1, L)]
            keep = ((skey >> 4) != (nkey >> 4)) & (skey < BIG)
            tgt = skey >> 4
            bvec = v * L + slane
            plsc.store_scatter(winner_v, [tgt], bvec, mask=keep)
            return c
        lax.fori_loop(0, B // L, scan_body, 0)

    @pl.when(wid < LW)
    def _labels():
        pltpu.sync_copy(y_in, y_v)
        l0 = wid * LROWS
        pltpu.sync_copy(lbl_in.at[pl.ds(l0, LROWS)], lbl_v)

        def lbl_body(v, c):
            wv = winner_v[pl.ds(l0 + v * L, L)]
            m = wv >= 0
            yv = plsc.load_gather(y_v, [jnp.maximum(wv, 0)])
            cur = lbl_v[pl.ds(v * L, L)]
            lbl_v[pl.ds(v * L, L)] = jnp.where(m, yv, cur)
            return c
        lax.fori_loop(0, LROWS // L, lbl_body, 0)
        pltpu.sync_copy(lbl_v, lbl_out.at[pl.ds(l0, LROWS)])

    @pl.when(wid == 0)
    def _compact():
        def cmp_body(g, base):
            wv = winner_v[pl.ds(g * L, L)]
            rowv = g * L + lane
            m = wv >= 0
            pc = plsc.cumsum(jnp.where(m, 1, 0))
            pos = base + pc - 1
            plsc.store_scatter(rows_l, [pos], rowv, mask=m)
            plsc.store_scatter(src_l, [pos], wv, mask=m)
            return base + pc[L - 1]
        cnt = lax.fori_loop(0, M // L, cmp_body, jnp.int32(0))

        @pl.when(cnt > 0)
        def _pad():
            lastrow = rows_l[pl.ds(cnt - 1, L)][0]
            lastsrc = src_l[pl.ds(cnt - 1, L)][0]
            rows_l[pl.ds(cnt, L)] = jnp.full((L,), lastrow, jnp.int32)
            src_l[pl.ds(cnt, L)] = jnp.full((L,), lastsrc, jnp.int32)

        @pl.when(cnt == 0)
        def _none():
            rows_l[pl.ds(0, L)] = jnp.zeros((L,), jnp.int32)
            src_l[pl.ds(0, L)] = jnp.zeros((L,), jnp.int32)

        pltpu.sync_copy(rows_l, rows_out)
        pltpu.sync_copy(src_l, src_out)
        shift_v[pl.ds(0, L)] = jnp.full((L,), cnt, jnp.int32)
        pltpu.sync_copy(shift_v.at[pl.ds(0, L)], cnt_out)


@functools.cache
def _build_sc():
    mesh = plsc.VectorSubcoreMesh(core_axis_name="c", subcore_axis_name="s",
                                  num_cores=NC, num_subcores=NS)
    return pl.kernel(
        _sc_body,
        out_type=(jax.ShapeDtypeStruct((M,), jnp.int32),
                  jax.ShapeDtypeStruct((LSZ,), jnp.int32),
                  jax.ShapeDtypeStruct((LSZ,), jnp.int32),
                  jax.ShapeDtypeStruct((L,), jnp.int32)),
        mesh=mesh,
        compiler_params=pltpu.CompilerParams(use_tc_tiling_on_sc=False,
                                             needs_layout_passes=False),
        scratch_types=dict(
            winner_v=pltpu.VMEM((M + L,), jnp.int32),
            idx_v=pltpu.VMEM((B,), jnp.int32),
            y_v=pltpu.VMEM((B,), jnp.int32),
            lbl_v=pltpu.VMEM((LROWS,), jnp.int32),
            shift_v=pltpu.VMEM((2 * L,), jnp.int32),
            rows_l=pltpu.VMEM((LSZ,), jnp.int32),
            src_l=pltpu.VMEM((LSZ,), jnp.int32),
        ),
    )


def _copy_body(src_ref, dst_ref):
    v = src_ref[...].reshape(D, CBT)
    dst_ref[...] = v.T


@functools.cache
def _build_copy():
    return pl.pallas_call(
        _copy_body,
        grid=((M + CBT - 1) // CBT,),
        in_specs=[pl.BlockSpec((3, 32, 32, CBT), lambda i: (0, 0, 0, i))],
        out_specs=pl.BlockSpec((CBT, D), lambda i: (i, 0)),
        out_shape=jax.ShapeDtypeStruct((M, D), jnp.float32),
    )


def _patch_body(rows_s, src_s, cnt_s, img_ref, x_ref, out_ref,
                bufs, gsems, osems):
    cnt = cnt_s[0]
    nch = (cnt + PS - 1) // PS

    def fire_gather(ch, grp):
        for s in range(PS):
            i = ch * PS + s

            @pl.when(i < cnt)
            def _g(i=i, s=s, grp=grp):
                w = src_s[i]
                pltpu.make_async_copy(
                    x_ref.at[w], bufs.at[grp * PS + s],
                    gsems.at[grp * PS + s]).start()

    def drain_gather_fire_write(ch, grp):
        for s in range(PS):
            i = ch * PS + s

            @pl.when(i < cnt)
            def _w(i=i, s=s, grp=grp):
                w = src_s[i]
                pltpu.make_async_copy(
                    x_ref.at[w], bufs.at[grp * PS + s],
                    gsems.at[grp * PS + s]).wait()
                r = rows_s[i]
                pltpu.make_async_copy(
                    bufs.at[grp * PS + s], out_ref.at[r],
                    osems.at[grp * PS + s]).start()

    def drain_write(ch, grp):
        for s in range(PS):
            i = ch * PS + s

            @pl.when(i < cnt)
            def _d(i=i, s=s, grp=grp):
                r = rows_s[i]
                pltpu.make_async_copy(
                    bufs.at[grp * PS + s], out_ref.at[r],
                    osems.at[grp * PS + s]).wait()

    def stage(ch, par):
        @pl.when(ch >= 2)
        def _a(ch=ch, par=par):
            drain_write(ch - 2, par)

        @pl.when(ch < nch)
        def _b(ch=ch, par=par):
            fire_gather(ch, par)

        @pl.when((ch >= 1) & (ch - 1 < nch))
        def _c(ch=ch, par=par):
            drain_gather_fire_write(ch - 1, 1 - par)

    def body(ch, c):
        p = lax.rem(ch, 2)

        @pl.when(p == 0)
        def _p0(ch=ch):
            stage(ch, 0)

        @pl.when(p == 1)
        def _p1(ch=ch):
            stage(ch, 1)
        return c
    lax.fori_loop(0, nch + 2, body, 0)


@functools.cache
def _build_patch():
    grid_spec = pltpu.PrefetchScalarGridSpec(
        num_scalar_prefetch=3,
        grid=(1,),
        in_specs=[
            pl.BlockSpec(memory_space=pl.ANY),
            pl.BlockSpec(memory_space=pl.ANY),
        ],
        out_specs=pl.BlockSpec(memory_space=pl.ANY),
        scratch_shapes=[
            pltpu.VMEM((2 * PS, D), jnp.float32),
            pltpu.SemaphoreType.DMA((2 * PS,)),
            pltpu.SemaphoreType.DMA((2 * PS,)),
        ],
    )
    return pl.pallas_call(
        _patch_body,
        grid_spec=grid_spec,
        out_shape=jax.ShapeDtypeStruct((M, D), jnp.float32),
        input_output_aliases={3: 0},
        compiler_params=pltpu.CompilerParams(
            has_side_effects=True),
    )


def kernel(buffer_img, buffer_label, x, y, idx):
    out_lbl, rows_l, src_l, cnt = _build_sc()(buffer_label, y, idx)
    copied = _build_copy()(jnp.transpose(buffer_img, (1, 2, 3, 0)))
    out_img = _build_patch()(rows_l, src_l, cnt, copied, x.reshape(B, D))
    return out_img.reshape(buffer_img.shape), out_lbl
